# baseline (speedup 1.0000x reference)
import functools

import jax
import jax.numpy as jnp
from jax import lax
from jax.experimental import pallas as pl
from jax.experimental.pallas import tpu as pltpu
from jax.experimental.pallas import tpu_sc as plsc

H = 4
D = 128
DH = D // H
_TYPES = ['user_turn', 'ai_turn', 'stance', 'pressure', 'belief']
_RELS = [('user_turn', 'asks', 'ai_turn'), ('ai_turn', 'responds', 'user_turn'),
         ('ai_turn', 'expresses', 'stance'), ('stance', 'shifts_to', 'stance'),
         ('pressure', 'applies_to', 'ai_turn'), ('ai_turn', 'about', 'belief')]
_NN = {'user_turn': 2500, 'ai_turn': 2500, 'stance': 2000, 'pressure': 1000,
       'belief': 2000}
_EC = {'asks': 40000, 'responds': 40000, 'expresses': 30000, 'shifts_to': 20000,
       'applies_to': 10000, 'about': 20000}

_DST_TYPES = ['user_turn', 'ai_turn', 'stance', 'belief']
_DST_OFF = {'user_turn': 0, 'ai_turn': 2500, 'stance': 5000, 'belief': 7000}
_NDST = 9000
_NPAD = 9216
_TRASH = _NDST

_SRC_OFF = {}
_off = 0
for (_s, _r, _d) in _RELS:
    _SRC_OFF[_r] = _off
    _off += _NN[_s]
_NSRC = _off

_ETOT = sum(_EC.values())
_NW = 32
_EPW = 5120
_EPAD = _NW * _EPW
_C = 32
_NCHUNK = _EPW // _C
_RPT = _NPAD // 16
_ND8 = 1536
_DRPT = _ND8 // 16


def _ln(x, g, b):
    m = x.mean(-1, keepdims=True)
    v = ((x - m) ** 2).mean(-1, keepdims=True)
    return (x - m) / jnp.sqrt(v + 1e-5) * g + b


def _linear_body(x_ref, w_ref, b_ref, o_ref):
    o_ref[...] = (jnp.dot(x_ref[...], w_ref[...],
                          preferred_element_type=jnp.float32) + b_ref[...])


def _plinear(x, W, b=None):
    N, K = x.shape
    F = W.shape[1]
    if b is None:
        b = jnp.zeros((F,), jnp.float32)
    return pl.pallas_call(
        _linear_body,
        out_shape=jax.ShapeDtypeStruct((N, F), jnp.float32),
    )(x, W, b.reshape(1, F))


def _fold_rel(W, A, scale=1.0):
    W4 = W.reshape(D, H, DH)
    return (jnp.einsum('ihd,hdf->ihf', W4, A) * scale).reshape(D, D)


def _edge_body(kt_hbm, mt_hbm, q_hbm, src_hbm, dst_hbm, dstd_hbm,
               num_hbm, den_hbm,
               sidx, didx, didx3, krows, qrows, mrows, outb, denb,
               num_sh, den_sh, sem0, sem1, sem2):
    cid = lax.axis_index("c")
    sid = lax.axis_index("s")
    wid = sid * 2 + cid
    lanes = lax.iota(jnp.int32, 16)

    zv = jnp.zeros((16,), jnp.float32)
    for e in range(_C):
        for j in range(D // 16):
            outb[e, pl.ds(j * 16, 16)] = zv
            denb[e, pl.ds(j * 16, 16)] = zv

    @pl.loop(0, _RPT // _C)
    def zinit(j):
        rb = sid * _RPT + j * _C
        pltpu.sync_copy(outb, num_sh.at[pl.ds(rb, _C)])

    @pl.loop(0, _DRPT // _C)
    def zinitd(j):
        rb = sid * _DRPT + j * _C
        pltpu.sync_copy(denb, den_sh.at[pl.ds(rb, _C)])

    plsc.subcore_barrier()

    ebase = wid * _EPW

    @pl.loop(0, _NCHUNK)
    def chunk(i):
        base = ebase + i * _C
        pltpu.sync_copy(src_hbm.at[pl.ds(base, _C)], sidx.at[0])
        pltpu.sync_copy(dst_hbm.at[pl.ds(base, _C)], didx.at[0])
        pltpu.sync_copy(dstd_hbm.at[pl.ds(base, _C)], didx3.at[0])
        ck = pltpu.async_copy(kt_hbm.at[sidx.at[0]], krows, sem0)
        cq = pltpu.async_copy(q_hbm.at[didx.at[0]], qrows, sem1)
        cm = pltpu.async_copy(mt_hbm.at[sidx.at[0]], mrows, sem2)
        ck.wait()
        cq.wait()
        cm.wait()
        perms = [lanes ^ s for s in (8, 4, 2, 1)]
        for g in range(_C // 16):
            dsel16 = ((didx[0, pl.ds(g * 16, 16)] & 7) * 16).astype(jnp.float32)
            for e16 in range(16):
                e = g * 16 + e16
                drow = zv
                for h in range(H):
                    lo = h * DH
                    hi = h * DH + 16
                    t = (qrows[e, pl.ds(lo, 16)] * krows[e, pl.ds(lo, 16)]
                         + qrows[e, pl.ds(hi, 16)] * krows[e, pl.ds(hi, 16)])
                    for p in perms:
                        t = t + t[p]
                    ex = jnp.exp(t)
                    outb[e, pl.ds(lo, 16)] = mrows[e, pl.ds(lo, 16)] * ex
                    outb[e, pl.ds(hi, 16)] = mrows[e, pl.ds(hi, 16)] * ex
                    drow = jnp.where(lanes == h, ex, drow)
                bsel = dsel16[jnp.full((16,), e16, jnp.int32)]
                for blk in range(8):
                    m = jnp.maximum(1.0 - jnp.abs(bsel - float(blk * 16)), 0.0)
                    denb[e, pl.ds(blk * 16, 16)] = drow * m
        pltpu.sync_copy(outb, num_sh.at[didx.at[0]], add=True)
        pltpu.sync_copy(denb, den_sh.at[didx3.at[0]], add=True)

    plsc.subcore_barrier()

    @pl.loop(0, _RPT // _C)
    def cpout(j):
        rb = sid * _RPT + j * _C
        pltpu.sync_copy(num_sh.at[pl.ds(rb, _C)], outb)
        pltpu.sync_copy(outb, num_hbm.at[cid, pl.ds(rb, _C)])

    @pl.loop(0, _DRPT // _C)
    def cpoutd(j):
        rb = sid * _DRPT + j * _C
        pltpu.sync_copy(den_sh.at[pl.ds(rb, _C)], denb)
        pltpu.sync_copy(denb, den_hbm.at[cid, pl.ds(rb, _C)])


@functools.cache
def _edge_call():
    return pl.kernel(
        _edge_body,
        mesh=plsc.VectorSubcoreMesh(core_axis_name="c", subcore_axis_name="s"),
        out_type=(jax.ShapeDtypeStruct((2, _NPAD, D), jnp.float32),
                  jax.ShapeDtypeStruct((2, _ND8, D), jnp.float32)),
        scratch_types=[
            pltpu.VMEM((1, _C), jnp.int32),
            pltpu.VMEM((1, _C), jnp.int32),
            pltpu.VMEM((1, _C), jnp.int32),
            pltpu.VMEM((_C, D), jnp.float32),
            pltpu.VMEM((_C, D), jnp.float32),
            pltpu.VMEM((_C, D), jnp.float32),
            pltpu.VMEM((_C, D), jnp.float32),
            pltpu.VMEM((_C, D), jnp.float32),
            pltpu.VMEM_SHARED((_NPAD, D), jnp.float32),
            pltpu.VMEM_SHARED((_ND8, D), jnp.float32),
            pltpu.SemaphoreType.DMA,
            pltpu.SemaphoreType.DMA,
            pltpu.SemaphoreType.DMA,
        ],
    )


def _edge_aggregate(KT, MT, Q, SRC, DST, DSTD):
    num2, den2 = _edge_call()(KT, MT, Q, SRC, DST, DSTD)
    num = (num2[0] + num2[1])[:_NDST]
    dent = (den2[0] + den2[1])[:_NPAD // 8]
    den = dent.reshape(_NPAD, 16)[:_NDST, :H]
    return num, den


def _build_edge_list(edges):
    srcs, dsts = [], []
    for (s, r, d) in _RELS:
        ei = edges[r].astype(jnp.int32)
        srcs.append(ei[0] + _SRC_OFF[r])
        dsts.append(ei[1] + _DST_OFF[d])
    SRC = jnp.concatenate(srcs)
    DST = jnp.concatenate(dsts)
    npad = _EPAD - _ETOT
    SRC = jnp.concatenate([SRC, jnp.zeros((npad,), jnp.int32)])
    DST = jnp.concatenate([DST, jnp.full((npad,), _TRASH, jnp.int32)])
    return SRC, DST, DST >> 3


def _hgt_layer(xd, SRC, DST, DSTD, lp):
    kts, mts = [], []
    for (s, r, d) in _RELS:
        kts.append(_plinear(xd[s], _fold_rel(lp['Wk'][s], lp['Aatt'][r],
                                             lp['mu'][r] / jnp.sqrt(float(DH)))))
        mts.append(_plinear(xd[s], _fold_rel(lp['Wv'][s], lp['Amsg'][r])))
    KT = jnp.concatenate(kts)
    MT = jnp.concatenate(mts)
    qs = [_plinear(xd[t], lp['Wq'][t]) for t in _DST_TYPES]
    Q = jnp.concatenate(qs + [jnp.zeros((_NPAD - _NDST, D), jnp.float32)])

    num, den = _edge_aggregate(KT, MT, Q, SRC, DST, DSTD)
    den_full = jnp.repeat(den + 1e-9, DH, axis=-1)
    agg = num / den_full

    out = {}
    for t in _TYPES:
        if t not in _DST_OFF:
            out[t] = xd[t]
            continue
        o = _DST_OFF[t]
        n = _NN[t]
        out[t] = xd[t] + _plinear(jax.nn.gelu(agg[o:o + n]), lp['Wa'][t])
    return out


def _mha(xq, xk, xv, p):
    Wi, bi = p['Wi'], p['bi']
    q = (xq @ Wi[:, :D] + bi[:D]).reshape(-1, H, DH).transpose(1, 0, 2)
    k = (xk @ Wi[:, D:2 * D] + bi[D:2 * D]).reshape(-1, H, DH).transpose(1, 0, 2)
    v = (xv @ Wi[:, 2 * D:] + bi[2 * D:]).reshape(-1, H, DH).transpose(1, 0, 2)
    s = jnp.einsum('hqd,hkd->hqk', q, k) / jnp.sqrt(float(DH))
    a = jax.nn.softmax(s, axis=-1)
    o = jnp.einsum('hqk,hkd->hqd', a, v).transpose(1, 0, 2).reshape(-1, D)
    return o @ p['Wo'] + p['bo']


def _forward_impl(xd, edges, params):
    SRC, DST, DSTD = _build_edge_list(edges)
    h = dict(xd)
    for li in range(2):
        h = _hgt_layer(h, SRC, DST, DSTD, params['hgt'][li])
        h = {t: _ln(jax.nn.gelu(h[t]), params['ln'][t]['g'], params['ln'][t]['b'])
             for t in _TYPES}
    user_h = h['user_turn']; ai_h = h['ai_turn']; stance_h = h['stance']; belief_h = h['belief']
    tp = params['traj']
    hs = jax.nn.gelu(_ln(_plinear(stance_h, tp['proj_W'], tp['proj_b']),
                         tp['proj_g'], tp['proj_be']))
    z = hs
    for lp in tp['trans']:
        qn = z @ lp['Wq']; kn = z @ lp['Wk']; v = z @ lp['Wv']
        qn = qn / (jnp.linalg.norm(qn, axis=-1, keepdims=True) + 1e-6)
        kn = kn / (jnp.linalg.norm(kn, axis=-1, keepdims=True) + 1e-6)
        num = qn @ (kn.T @ v) + v.sum(0)
        den = qn @ kn.sum(0) + float(z.shape[0])
        z = z + (num / den[:, None]) @ lp['Wo']
    hm = 0.7 * hs + 0.3 * z
    traj_summary = _mha(hm, hm, hm, tp['mha']).mean(0, keepdims=True)
    xc = hm.T[None]
    dn = ('NCH', 'OIH', 'NCH')
    c1 = jax.nn.gelu(jax.lax.conv_general_dilated(xc, tp['conv1_W'], (1,), 'SAME',
                                                  dimension_numbers=dn)
                     + tp['conv1_b'][None, :, None])
    c2 = jax.nn.gelu(jax.lax.conv_general_dilated(c1, tp['conv2_W'], (1,), 'SAME',
                                                  dimension_numbers=dn)
                     + tp['conv2_b'][None, :, None])
    decay_summary = c2.mean(2)
    traj_emb = jnp.concatenate([traj_summary, decay_summary], -1) @ tp['out_W'] + tp['out_b']
    pp = params['press']
    ai_ctx = _mha(ai_h, user_h, user_h, pp['mha'])
    mlen = min(ai_h.shape[0], user_h.shape[0])
    comb = jnp.concatenate([ai_h[:mlen], user_h[:mlen]], -1)
    pressure_scores = jax.nn.sigmoid(
        (jax.nn.relu(comb @ pp['s1_W'] + pp['s1_b']) @ pp['s2_W'] + pp['s2_b'])[:, 0])
    ai_pooled = ai_ctx.mean(0, keepdims=True)
    belief_pooled = _mha(ai_h, belief_h, belief_h, params['belief_mha']).mean(0, keepdims=True)
    cp = params['cmp']
    scmp = jnp.concatenate([stance_h[:1], stance_h[-1:]], -1)
    scmp = jax.nn.relu(scmp @ cp['W1'] + cp['b1']) @ cp['W2'] + cp['b2']
    cf = params['clf']
    ci = jnp.concatenate([traj_emb, ai_pooled, belief_pooled, scmp], -1)
    hc = jax.nn.relu(_ln(ci @ cf['W1'] + cf['b1'], cf['g'], cf['be']))
    hc = jax.nn.relu(hc @ cf['W2'] + cf['b2'])
    logits = (hc @ cf['W3'] + cf['b3']).reshape(-1)
    decay = jax.nn.sigmoid(logits)
    per_turn = jax.nn.sigmoid(ai_ctx @ params['turn_W'] + params['turn_b'])[:, 0]
    return jnp.concatenate([logits, decay, per_turn, pressure_scores])


def kernel(x_user_turn, x_ai_turn, x_stance, x_pressure, x_belief,
           edge_asks, edge_responds, edge_expresses, edge_shifts_to,
           edge_applies_to, edge_about, params):
    xd = {'user_turn': x_user_turn, 'ai_turn': x_ai_turn, 'stance': x_stance,
          'pressure': x_pressure, 'belief': x_belief}
    edges = {'asks': edge_asks, 'responds': edge_responds,
             'expresses': edge_expresses, 'shifts_to': edge_shifts_to,
             'applies_to': edge_applies_to, 'about': edge_about}
    return _forward_impl(xd, edges, params)


# speedup vs baseline: 1.0879x; 1.0879x over previous
import functools

import jax
import jax.numpy as jnp
from jax import lax
from jax.experimental import pallas as pl
from jax.experimental.pallas import tpu as pltpu
from jax.experimental.pallas import tpu_sc as plsc

H = 4
D = 128
DH = D // H
_TYPES = ['user_turn', 'ai_turn', 'stance', 'pressure', 'belief']
_RELS = [('user_turn', 'asks', 'ai_turn'), ('ai_turn', 'responds', 'user_turn'),
         ('ai_turn', 'expresses', 'stance'), ('stance', 'shifts_to', 'stance'),
         ('pressure', 'applies_to', 'ai_turn'), ('ai_turn', 'about', 'belief')]
_NN = {'user_turn': 2500, 'ai_turn': 2500, 'stance': 2000, 'pressure': 1000,
       'belief': 2000}
_EC = {'asks': 40000, 'responds': 40000, 'expresses': 30000, 'shifts_to': 20000,
       'applies_to': 10000, 'about': 20000}

_DST_TYPES = ['user_turn', 'ai_turn', 'stance', 'belief']
_DST_OFF = {'user_turn': 0, 'ai_turn': 2500, 'stance': 5000, 'belief': 7000}
_NDST = 9000
_NPAD = 9216
_TRASH = _NDST

_SRC_OFF = {}
_off = 0
for (_s, _r, _d) in _RELS:
    _SRC_OFF[_r] = _off
    _off += _NN[_s]
_NSRC = _off

_ETOT = sum(_EC.values())
_NW = 32
_EPW = 5120
_EPAD = _NW * _EPW
_C = 32
_CZ = 32
_NCHUNK = _EPW // _C
_RPT = _NPAD // 16
_ND8 = 1536
_DRPT = _ND8 // 16


def _ln(x, g, b):
    m = x.mean(-1, keepdims=True)
    v = ((x - m) ** 2).mean(-1, keepdims=True)
    return (x - m) / jnp.sqrt(v + 1e-5) * g + b


def _linear_body(x_ref, w_ref, b_ref, o_ref):
    o_ref[...] = (jnp.dot(x_ref[...], w_ref[...],
                          preferred_element_type=jnp.float32) + b_ref[...])


def _plinear(x, W, b=None):
    N, K = x.shape
    F = W.shape[1]
    if b is None:
        b = jnp.zeros((F,), jnp.float32)
    return pl.pallas_call(
        _linear_body,
        out_shape=jax.ShapeDtypeStruct((N, F), jnp.float32),
    )(x, W, b.reshape(1, F))


def _fold_rel(W, A, scale=1.0):
    W4 = W.reshape(D, H, DH)
    return (jnp.einsum('ihd,hdf->ihf', W4, A) * scale).reshape(D, D)


def _edge_body(kt_hbm, mt_hbm, q_hbm, cidx_hbm,
               num_hbm, den_hbm,
               cidx, krows, qrows, mrows, outb, denb,
               num_sh, den_sh, sem0, sem1, sem2):
    cid = lax.axis_index("c")
    sid = lax.axis_index("s")
    wid = sid * 2 + cid
    lanes = lax.iota(jnp.int32, 16)

    zv = jnp.zeros((16,), jnp.float32)
    for e in range(_C):
        for j in range(D // 16):
            outb[e, pl.ds(j * 16, 16)] = zv
            denb[e, pl.ds(j * 16, 16)] = zv

    @pl.loop(0, _RPT // _C)
    def zinit(j):
        rb = sid * _RPT + j * _C
        pltpu.sync_copy(outb, num_sh.at[pl.ds(rb, _C)])

    @pl.loop(0, _DRPT // _CZ)
    def zinitd(j):
        rb = sid * _DRPT + j * _CZ
        pltpu.sync_copy(denb.at[pl.ds(0, _CZ)], den_sh.at[pl.ds(rb, _CZ)])

    plsc.subcore_barrier()

    cbase = wid * _NCHUNK

    @pl.loop(0, _NCHUNK)
    def chunk(i):
        pltpu.sync_copy(cidx_hbm.at[cbase + i], cidx)
        ck = pltpu.async_copy(kt_hbm.at[cidx.at[0]], krows, sem0)
        cq = pltpu.async_copy(q_hbm.at[cidx.at[1]], qrows, sem1)
        cm = pltpu.async_copy(mt_hbm.at[cidx.at[0]], mrows, sem2)
        ck.wait()
        cq.wait()
        cm.wait()
        perms = [lanes ^ s for s in (8, 4, 2, 1)]
        for g in range(_C // 16):
            dsel16 = ((cidx[1, pl.ds(g * 16, 16)] & 7) * 16).astype(jnp.float32)
            for e16 in range(16):
                e = g * 16 + e16
                drow = zv
                for h in range(H):
                    lo = h * DH
                    hi = h * DH + 16
                    t = (qrows[e, pl.ds(lo, 16)] * krows[e, pl.ds(lo, 16)]
                         + qrows[e, pl.ds(hi, 16)] * krows[e, pl.ds(hi, 16)])
                    for p in perms:
                        t = t + t[p]
                    ex = jnp.exp(t)
                    outb[e, pl.ds(lo, 16)] = mrows[e, pl.ds(lo, 16)] * ex
                    outb[e, pl.ds(hi, 16)] = mrows[e, pl.ds(hi, 16)] * ex
                    drow = jnp.where(lanes == h, ex, drow)
                bsel = dsel16[jnp.full((16,), e16, jnp.int32)]
                for blk in range(8):
                    m = jnp.maximum(1.0 - jnp.abs(bsel - float(blk * 16)), 0.0)
                    denb[e, pl.ds(blk * 16, 16)] = drow * m
        pltpu.sync_copy(outb, num_sh.at[cidx.at[1]], add=True)
        pltpu.sync_copy(denb, den_sh.at[cidx.at[2]], add=True)

    plsc.subcore_barrier()

    @pl.loop(0, _RPT // _C)
    def cpout(j):
        rb = sid * _RPT + j * _C
        pltpu.sync_copy(num_sh.at[pl.ds(rb, _C)], outb)
        pltpu.sync_copy(outb, num_hbm.at[cid, pl.ds(rb, _C)])

    @pl.loop(0, _DRPT // _CZ)
    def cpoutd(j):
        rb = sid * _DRPT + j * _CZ
        pltpu.sync_copy(den_sh.at[pl.ds(rb, _CZ)], denb.at[pl.ds(0, _CZ)])
        pltpu.sync_copy(denb.at[pl.ds(0, _CZ)], den_hbm.at[cid, pl.ds(rb, _CZ)])


@functools.cache
def _edge_call():
    return pl.kernel(
        _edge_body,
        mesh=plsc.VectorSubcoreMesh(core_axis_name="c", subcore_axis_name="s"),
        out_type=(jax.ShapeDtypeStruct((2, _NPAD, D), jnp.float32),
                  jax.ShapeDtypeStruct((2, _ND8, D), jnp.float32)),
        scratch_types=[
            pltpu.VMEM((3, _C), jnp.int32),
            pltpu.VMEM((_C, D), jnp.float32),
            pltpu.VMEM((_C, D), jnp.float32),
            pltpu.VMEM((_C, D), jnp.float32),
            pltpu.VMEM((_C, D), jnp.float32),
            pltpu.VMEM((_C, D), jnp.float32),
            pltpu.VMEM_SHARED((_NPAD, D), jnp.float32),
            pltpu.VMEM_SHARED((_ND8, D), jnp.float32),
            pltpu.SemaphoreType.DMA,
            pltpu.SemaphoreType.DMA,
            pltpu.SemaphoreType.DMA,
        ],
    )


def _edge_aggregate(KT, MT, Q, CIDX):
    num2, den2 = _edge_call()(KT, MT, Q, CIDX)
    num = (num2[0] + num2[1])[:_NDST]
    dent = (den2[0] + den2[1])[:_NPAD // 8]
    den = dent.reshape(_NPAD, 16)[:_NDST, :H]
    return num, den


def _build_edge_list(edges):
    srcs, dsts = [], []
    for (s, r, d) in _RELS:
        ei = edges[r].astype(jnp.int32)
        srcs.append(ei[0] + _SRC_OFF[r])
        dsts.append(ei[1] + _DST_OFF[d])
    SRC = jnp.concatenate(srcs)
    DST = jnp.concatenate(dsts)
    npad = _EPAD - _ETOT
    SRC = jnp.concatenate([SRC, jnp.zeros((npad,), jnp.int32)])
    DST = jnp.concatenate([DST, jnp.full((npad,), _TRASH, jnp.int32)])
    return jnp.stack([SRC.reshape(-1, _C), DST.reshape(-1, _C),
                      (DST >> 3).reshape(-1, _C)], axis=1)


def _hgt_layer(xd, CIDX, lp):
    kts, mts = [], []
    for (s, r, d) in _RELS:
        kts.append(_plinear(xd[s], _fold_rel(lp['Wk'][s], lp['Aatt'][r],
                                             lp['mu'][r] / jnp.sqrt(float(DH)))))
        mts.append(_plinear(xd[s], _fold_rel(lp['Wv'][s], lp['Amsg'][r])))
    KT = jnp.concatenate(kts)
    MT = jnp.concatenate(mts)
    qs = [_plinear(xd[t], lp['Wq'][t]) for t in _DST_TYPES]
    Q = jnp.concatenate(qs + [jnp.zeros((_NPAD - _NDST, D), jnp.float32)])

    num, den = _edge_aggregate(KT, MT, Q, CIDX)
    den_full = jnp.repeat(den + 1e-9, DH, axis=-1)
    agg = num / den_full

    out = {}
    for t in _TYPES:
        if t not in _DST_OFF:
            out[t] = xd[t]
            continue
        o = _DST_OFF[t]
        n = _NN[t]
        out[t] = xd[t] + _plinear(jax.nn.gelu(agg[o:o + n]), lp['Wa'][t])
    return out


def _mha(xq, xk, xv, p):
    Wi, bi = p['Wi'], p['bi']
    q = (xq @ Wi[:, :D] + bi[:D]).reshape(-1, H, DH).transpose(1, 0, 2)
    k = (xk @ Wi[:, D:2 * D] + bi[D:2 * D]).reshape(-1, H, DH).transpose(1, 0, 2)
    v = (xv @ Wi[:, 2 * D:] + bi[2 * D:]).reshape(-1, H, DH).transpose(1, 0, 2)
    s = jnp.einsum('hqd,hkd->hqk', q, k) / jnp.sqrt(float(DH))
    a = jax.nn.softmax(s, axis=-1)
    o = jnp.einsum('hqk,hkd->hqd', a, v).transpose(1, 0, 2).reshape(-1, D)
    return o @ p['Wo'] + p['bo']


def _forward_impl(xd, edges, params):
    CIDX = _build_edge_list(edges)
    h = dict(xd)
    for li in range(2):
        h = _hgt_layer(h, CIDX, params['hgt'][li])
        h = {t: _ln(jax.nn.gelu(h[t]), params['ln'][t]['g'], params['ln'][t]['b'])
             for t in _TYPES}
    user_h = h['user_turn']; ai_h = h['ai_turn']; stance_h = h['stance']; belief_h = h['belief']
    tp = params['traj']
    hs = jax.nn.gelu(_ln(_plinear(stance_h, tp['proj_W'], tp['proj_b']),
                         tp['proj_g'], tp['proj_be']))
    z = hs
    for lp in tp['trans']:
        qn = z @ lp['Wq']; kn = z @ lp['Wk']; v = z @ lp['Wv']
        qn = qn / (jnp.linalg.norm(qn, axis=-1, keepdims=True) + 1e-6)
        kn = kn / (jnp.linalg.norm(kn, axis=-1, keepdims=True) + 1e-6)
        num = qn @ (kn.T @ v) + v.sum(0)
        den = qn @ kn.sum(0) + float(z.shape[0])
        z = z + (num / den[:, None]) @ lp['Wo']
    hm = 0.7 * hs + 0.3 * z
    traj_summary = _mha(hm, hm, hm, tp['mha']).mean(0, keepdims=True)
    xc = hm.T[None]
    dn = ('NCH', 'OIH', 'NCH')
    c1 = jax.nn.gelu(jax.lax.conv_general_dilated(xc, tp['conv1_W'], (1,), 'SAME',
                                                  dimension_numbers=dn)
                     + tp['conv1_b'][None, :, None])
    c2 = jax.nn.gelu(jax.lax.conv_general_dilated(c1, tp['conv2_W'], (1,), 'SAME',
                                                  dimension_numbers=dn)
                     + tp['conv2_b'][None, :, None])
    decay_summary = c2.mean(2)
    traj_emb = jnp.concatenate([traj_summary, decay_summary], -1) @ tp['out_W'] + tp['out_b']
    pp = params['press']
    ai_ctx = _mha(ai_h, user_h, user_h, pp['mha'])
    mlen = min(ai_h.shape[0], user_h.shape[0])
    comb = jnp.concatenate([ai_h[:mlen], user_h[:mlen]], -1)
    pressure_scores = jax.nn.sigmoid(
        (jax.nn.relu(comb @ pp['s1_W'] + pp['s1_b']) @ pp['s2_W'] + pp['s2_b'])[:, 0])
    ai_pooled = ai_ctx.mean(0, keepdims=True)
    belief_pooled = _mha(ai_h, belief_h, belief_h, params['belief_mha']).mean(0, keepdims=True)
    cp = params['cmp']
    scmp = jnp.concatenate([stance_h[:1], stance_h[-1:]], -1)
    scmp = jax.nn.relu(scmp @ cp['W1'] + cp['b1']) @ cp['W2'] + cp['b2']
    cf = params['clf']
    ci = jnp.concatenate([traj_emb, ai_pooled, belief_pooled, scmp], -1)
    hc = jax.nn.relu(_ln(ci @ cf['W1'] + cf['b1'], cf['g'], cf['be']))
    hc = jax.nn.relu(hc @ cf['W2'] + cf['b2'])
    logits = (hc @ cf['W3'] + cf['b3']).reshape(-1)
    decay = jax.nn.sigmoid(logits)
    per_turn = jax.nn.sigmoid(ai_ctx @ params['turn_W'] + params['turn_b'])[:, 0]
    return jnp.concatenate([logits, decay, per_turn, pressure_scores])


def kernel(x_user_turn, x_ai_turn, x_stance, x_pressure, x_belief,
           edge_asks, edge_responds, edge_expresses, edge_shifts_to,
           edge_applies_to, edge_about, params):
    xd = {'user_turn': x_user_turn, 'ai_turn': x_ai_turn, 'stance': x_stance,
          'pressure': x_pressure, 'belief': x_belief}
    edges = {'asks': edge_asks, 'responds': edge_responds,
             'expresses': edge_expresses, 'shifts_to': edge_shifts_to,
             'applies_to': edge_applies_to, 'about': edge_about}
    return _forward_impl(xd, edges, params)
